# Initial kernel scaffold; baseline (speedup 1.0000x reference)
import jax
import jax.numpy as jnp
from jax.experimental import pallas as pl


def kernel(user_emb, item_emb, edge_index, edge_weight):
    raise NotImplementedError("write your pallas kernel here")


# speedup vs baseline: 8.8506x; 8.8506x over previous
import functools

import jax
import jax.numpy as jnp
from jax import lax
from jax.experimental import pallas as pl
from jax.experimental.pallas import tpu as pltpu
from jax.experimental.pallas import tpu_sc as plsc

NN = 10000
D = 128
NE = 320000
NLAYERS = 3
NC = 2
NS = 16
EPC = NE // NC
EPT = EPC // NS
K = 64
NCHUNK = EPT // K
KREM = EPT - NCHUNK * K
RPT = 624
RTAIL = NN - RPT * NS


NPAIR = NCHUNK // 2


def _scale_rows(rbuf, w_v, wbase, k):
    def group_body(g, carry):
        wv16 = w_v[pl.ds(wbase + g * 16, 16)]
        for j2 in range(16):
            wj = wv16[j2]
            for v in range(D // 16):
                sl = pl.ds(v * 16, 16)
                rbuf[g * 16 + j2, sl] = rbuf[g * 16 + j2, sl] * wj
        return carry

    lax.fori_loop(0, k // 16, group_body, 0, unroll=False)


def _spmm_body(e_hbm, row_hbm, col_hbm, w_hbm, z_hbm, out_hbm,
               col_v, w_v, rA, rB, rbA, rbB, row_t, rows_t, acc,
               gsA, gsB, ssA, ssB, rsA, rsB, ts):
    c = lax.axis_index("c")
    s = lax.axis_index("s")

    rbase = s * RPT
    pltpu.sync_copy(z_hbm.at[pl.ds(0, RPT)], acc.at[pl.ds(rbase, RPT)])

    @pl.when(s == 0)
    def _():
        pltpu.sync_copy(z_hbm.at[pl.ds(0, RTAIL)], acc.at[pl.ds(RPT * NS, RTAIL)])

    ebase = c * EPC + s * EPT
    pltpu.sync_copy(col_hbm.at[pl.ds(ebase, EPT)], col_v)
    pltpu.sync_copy(w_hbm.at[pl.ds(ebase, EPT)], w_v)
    plsc.subcore_barrier()

    def row_src(g):
        return row_hbm.at[pl.ds(ebase + g * K, K)]

    def gather_src(g):
        return e_hbm.at[col_v.at[pl.ds(g * K, K)]]

    def issue(g, rref, rbuf, gsem, rsem):
        pltpu.async_copy(row_src(g), rref, rsem)
        pltpu.async_copy(gather_src(g), rbuf, gsem)

    def wait_gather(g, rbuf, gsem):
        pltpu.make_async_copy(gather_src(g), rbuf, gsem).wait()

    def do_scatter(g, rref, rbuf, rsem, ssem):
        pltpu.make_async_copy(row_src(g), rref, rsem).wait()
        pltpu.async_copy(rbuf, acc.at[rref], ssem, add=True)

    def wait_scatter(rref, rbuf, ssem):
        pltpu.make_async_copy(rbuf, acc.at[rref], ssem).wait()

    issue(0, rA, rbA, gsA, rsA)

    def pair_body(p, carry):
        g = 2 * p

        @pl.when(p > 0)
        def _():
            wait_scatter(rB, rbB, ssB)

        issue(g + 1, rB, rbB, gsB, rsB)
        wait_gather(g, rbA, gsA)
        _scale_rows(rbA, w_v, g * K, K)
        do_scatter(g, rA, rbA, rsA, ssA)
        wait_gather(g + 1, rbB, gsB)
        _scale_rows(rbB, w_v, (g + 1) * K, K)
        wait_scatter(rA, rbA, ssA)

        @pl.when(p < NPAIR - 1)
        def _():
            issue(g + 2, rA, rbA, gsA, rsA)

        do_scatter(g + 1, rB, rbB, rsB, ssB)
        return carry

    lax.fori_loop(0, NPAIR, pair_body, 0, unroll=False)
    wait_scatter(rB, rbB, ssB)

    toff = ebase + NCHUNK * K
    pltpu.sync_copy(row_hbm.at[pl.ds(toff, KREM)], row_t)
    pltpu.async_copy(e_hbm.at[col_v.at[pl.ds(NCHUNK * K, KREM)]], rows_t, ts).wait()
    _scale_rows(rows_t, w_v, NCHUNK * K, KREM)
    pltpu.sync_copy(rows_t, acc.at[row_t], add=True)

    plsc.subcore_barrier()
    pltpu.sync_copy(acc.at[pl.ds(rbase, RPT)], out_hbm.at[c, pl.ds(rbase, RPT)])

    @pl.when(s == 0)
    def _():
        pltpu.sync_copy(acc.at[pl.ds(RPT * NS, RTAIL)],
                        out_hbm.at[c, pl.ds(RPT * NS, RTAIL)])


@functools.cache
def _make_spmm(interpret: bool = False):
  return pl.kernel(
    _spmm_body,
    out_type=jax.ShapeDtypeStruct((NC, NN, D), jnp.float32),
    mesh=plsc.VectorSubcoreMesh(
        core_axis_name="c", subcore_axis_name="s", num_cores=NC, num_subcores=NS
    ),
    interpret=interpret,
    scratch_types=[
        pltpu.VMEM((EPT,), jnp.int32),
        pltpu.VMEM((EPT,), jnp.float32),
        pltpu.VMEM((K,), jnp.int32),
        pltpu.VMEM((K,), jnp.int32),
        pltpu.VMEM((K, D), jnp.float32),
        pltpu.VMEM((K, D), jnp.float32),
        pltpu.VMEM((KREM,), jnp.int32),
        pltpu.VMEM((KREM, D), jnp.float32),
        pltpu.VMEM_SHARED((NN, D), jnp.float32),
        pltpu.SemaphoreType.DMA,
        pltpu.SemaphoreType.DMA,
        pltpu.SemaphoreType.DMA,
        pltpu.SemaphoreType.DMA,
        pltpu.SemaphoreType.DMA,
        pltpu.SemaphoreType.DMA,
        pltpu.SemaphoreType.DMA,
    ],
  )


def _combine_body(p_ref, s_ref, e_ref, sn_ref, m_ref):
    e = p_ref[0] + p_ref[1]
    sn = s_ref[...] + e
    e_ref[...] = e
    sn_ref[...] = sn
    m_ref[...] = sn * 0.25


_RB = 1000

_combine = pl.pallas_call(
    _combine_body,
    grid=(NN // _RB,),
    in_specs=[
        pl.BlockSpec((NC, _RB, D), lambda i: (0, i, 0)),
        pl.BlockSpec((_RB, D), lambda i: (i, 0)),
    ],
    out_specs=[pl.BlockSpec((_RB, D), lambda i: (i, 0))] * 3,
    out_shape=[jax.ShapeDtypeStruct((NN, D), jnp.float32)] * 3,
)


@jax.jit
def kernel(user_emb, item_emb, edge_index, edge_weight):
    e = jnp.concatenate([user_emb, item_emb], axis=0)
    row = edge_index[0]
    col = edge_index[1]
    z = jnp.zeros((RPT, D), jnp.float32)
    s = e
    m = None
    spmm = _make_spmm()
    for _ in range(NLAYERS):
        p = spmm(e, row, col, edge_weight, z)
        e, s, m = _combine(p, s)
    return m[:NN // 2], m[NN // 2:]
